# baseline (speedup 1.0000x reference)
import jax
import jax.numpy as jnp
from jax import lax
from jax.experimental import pallas as pl
from jax.experimental.pallas import tpu as pltpu
from jax.experimental.pallas import tpu_sc as plsc

N = 10000
E = 320000
D = 128

NC = 2
NS = 16
B = 64
IDXR = 80
CPT = 2 * IDXR
EROWS = NC * NS * IDXR
EPAD = EROWS * 128
ROWS_PER_TILE = 624
TAIL_ROWS = N - NS * ROWS_PER_TILE


def _sc_body(x_hbm, src_hbm, dst_hbm, attn_hbm, hp_hbm,
             acc, srcb, dstb, attnb, rows,
             sem_i, sem_g0, sem_g1, sem_s0, sem_s1):
    sem_g = (sem_g0, sem_g1)
    sem_s = (sem_s0, sem_s1)
    c = lax.axis_index("c")
    s = lax.axis_index("s")

    r0 = (c * NS + s) * IDXR
    pltpu.async_copy(src_hbm.at[pl.ds(r0, IDXR)], srcb, sem_i)
    pltpu.async_copy(dst_hbm.at[pl.ds(r0, IDXR)], dstb, sem_i)
    pltpu.async_copy(attn_hbm.at[pl.ds(r0, IDXR)], attnb, sem_i)

    def zrow(i, carry):
        for j in range(D // 16):
            rows[0, i, pl.ds(j * 16, 16)] = jnp.zeros((16,), jnp.float32)
        return carry

    lax.fori_loop(0, B, zrow, 0)
    for q in range(ROWS_PER_TILE // B):
        pltpu.sync_copy(rows.at[0],
                        acc.at[pl.ds(s * ROWS_PER_TILE + q * B, B)])
    rem = ROWS_PER_TILE - (ROWS_PER_TILE // B) * B
    pltpu.sync_copy(
        rows.at[0, pl.ds(0, rem)],
        acc.at[pl.ds(s * ROWS_PER_TILE + ROWS_PER_TILE - rem, rem)])

    @pl.when(s == NS - 1)
    def _zero_tail():
        pltpu.sync_copy(rows.at[0, pl.ds(0, TAIL_ROWS)],
                        acc.at[pl.ds(NS * ROWS_PER_TILE, TAIL_ROWS)])

    plsc.subcore_barrier()

    pltpu.make_async_copy(src_hbm.at[pl.ds(r0, IDXR)], srcb, sem_i).wait()
    pltpu.make_async_copy(dst_hbm.at[pl.ds(r0, IDXR)], dstb, sem_i).wait()
    pltpu.make_async_copy(attn_hbm.at[pl.ds(r0, IDXR)], attnb, sem_i).wait()

    def issue_gather(jj, h):
        pltpu.async_copy(x_hbm.at[srcb.at[jj, pl.ds(h * B, B)]],
                         rows.at[h], sem_g[h])

    def wait_gather(jj, h):
        pltpu.make_async_copy(x_hbm.at[srcb.at[jj, pl.ds(h * B, B)]],
                              rows.at[h], sem_g[h]).wait()

    def issue_scatter(jj, h):
        pltpu.async_copy(rows.at[h], acc.at[dstb.at[jj, pl.ds(h * B, B)]],
                         sem_s[h], add=True)

    def wait_scatter(jj, h):
        pltpu.make_async_copy(rows.at[h],
                              acc.at[dstb.at[jj, pl.ds(h * B, B)]],
                              sem_s[h]).wait()

    def scale(jj, h):
        def rowscale(g, rcarry):
            av = attnb[jj, pl.ds(h * B + g * 16, 16)]
            for t in range(16):
                a = jnp.full((16,), av[t], jnp.float32)
                for f in range(D // 16):
                    rows[h, g * 16 + t, pl.ds(f * 16, 16)] = (
                        rows[h, g * 16 + t, pl.ds(f * 16, 16)] * a)
            return rcarry

        lax.fori_loop(0, B // 16, rowscale, 0)

    issue_gather(0, 0)
    issue_gather(0, 1)
    wait_gather(0, 0)
    scale(0, 0)
    issue_scatter(0, 0)

    def loop_body(i, carry):
        wait_scatter(i, 0)
        issue_gather(i + 1, 0)
        wait_gather(i, 1)
        scale(i, 1)
        issue_scatter(i, 1)

        wait_scatter(i, 1)
        issue_gather(i + 1, 1)
        wait_gather(i + 1, 0)
        scale(i + 1, 0)
        issue_scatter(i + 1, 0)
        return carry

    lax.fori_loop(0, IDXR - 1, loop_body, 0)
    wait_scatter(IDXR - 1, 0)
    wait_gather(IDXR - 1, 1)
    scale(IDXR - 1, 1)
    issue_scatter(IDXR - 1, 1)
    wait_scatter(IDXR - 1, 1)
    plsc.subcore_barrier()

    pltpu.sync_copy(acc.at[pl.ds(s * ROWS_PER_TILE, ROWS_PER_TILE)],
                    hp_hbm.at[c, pl.ds(s * ROWS_PER_TILE, ROWS_PER_TILE)])

    @pl.when(s == NS - 1)
    def _drain_tail():
        pltpu.sync_copy(acc.at[pl.ds(NS * ROWS_PER_TILE, TAIL_ROWS)],
                        hp_hbm.at[c, pl.ds(NS * ROWS_PER_TILE, TAIL_ROWS)])


def _sc_message_passing(x, src2, dst2, attn2):
    mesh = plsc.VectorSubcoreMesh(core_axis_name="c", subcore_axis_name="s")
    kern = pl.kernel(
        _sc_body,
        mesh=mesh,
        out_type=jax.ShapeDtypeStruct((NC, N, D), jnp.float32),
        scratch_types=[
            pltpu.VMEM_SHARED((N, D), jnp.float32),
            pltpu.VMEM((IDXR, 128), jnp.int32),
            pltpu.VMEM((IDXR, 128), jnp.int32),
            pltpu.VMEM((IDXR, 128), jnp.float32),
            pltpu.VMEM((2, B, D), jnp.float32),
            pltpu.SemaphoreType.DMA,
            pltpu.SemaphoreType.DMA,
            pltpu.SemaphoreType.DMA,
            pltpu.SemaphoreType.DMA,
            pltpu.SemaphoreType.DMA,
        ],
    )
    return kern(x, src2, dst2, attn2)


def _tc_body(x_ref, h0_ref, h1_ref, w1_ref, b1_ref, w2_ref, b2_ref, o_ref):
    x = x_ref[...]
    hn = h0_ref[0] + h1_ref[0]
    u = x + hn
    v = x * hn
    dn = (((1,), (1,)), ((), ()))
    y1 = lax.dot_general(u, w1_ref[...], dn,
                         preferred_element_type=jnp.float32) + b1_ref[...]
    y1 = jnp.where(y1 >= 0, y1, y1 * 0.01)
    y2 = lax.dot_general(v, w2_ref[...], dn,
                         preferred_element_type=jnp.float32) + b2_ref[...]
    y2 = jnp.where(y2 >= 0, y2, y2 * 0.01)
    o_ref[...] = y1 + y2


def _tc_dense(x, hp, W1, b1, W2, b2):
    BN = 1000
    grid = (N // BN,)
    row_spec = pl.BlockSpec((BN, D), lambda i: (i, 0))
    h0_spec = pl.BlockSpec((1, BN, D), lambda i: (0, i, 0))
    h1_spec = pl.BlockSpec((1, BN, D), lambda i: (1, i, 0))
    full_spec = pl.BlockSpec((D, D), lambda i: (0, 0))
    bias_spec = pl.BlockSpec((1, D), lambda i: (0, 0))
    return pl.pallas_call(
        _tc_body,
        grid=grid,
        in_specs=[row_spec, h0_spec, h1_spec, full_spec, bias_spec,
                  full_spec, bias_spec],
        out_specs=row_spec,
        out_shape=jax.ShapeDtypeStruct((N, D), jnp.float32),
    )(x, hp, hp, W1, b1, W2, b2)


@jax.jit
def kernel(x, edge_index, edge_attn, W1, b1, W2, b2):
    pad = EPAD - E
    src2 = jnp.concatenate(
        [edge_index[0], jnp.zeros((pad,), jnp.int32)]).reshape(EROWS, 128)
    dst2 = jnp.concatenate(
        [edge_index[1], jnp.zeros((pad,), jnp.int32)]).reshape(EROWS, 128)
    attn2 = jnp.concatenate(
        [edge_attn.reshape(E), jnp.zeros((pad,), jnp.float32)]
    ).reshape(EROWS, 128)
    hp = _sc_message_passing(x, src2, dst2, attn2)
    out = _tc_dense(x, hp, W1, b1.reshape(1, D), W2, b2.reshape(1, D))
    return out
